# baseline (speedup 1.0000x reference)
import functools

import jax
import jax.numpy as jnp
from jax import lax
from jax.experimental import pallas as pl
from jax.experimental.pallas import tpu as pltpu
from jax.experimental.pallas import tpu_sc as plsc

_B = 16384
_D = 64
_NC = 2
_NS = 16
_NW = _NC * _NS
_BPW = _B // _NW
_CHUNK = 128
_NCHUNK = _BPW // _CHUNK
_LANES = 16


def _bpr_body(user_h, item_i_h, item_j_h, uw_h, iw_h, out_i_h, out_j_h,
              idx_u, idx_i, idx_j, rows_u, rows_i, rows_j,
              out_i_v, out_j_v, sem):
    c = lax.axis_index("c")
    s = lax.axis_index("s")
    wid = s * _NC + c
    base = wid * _BPW
    chunk_base = wid * _NCHUNK

    pltpu.sync_copy(user_h.at[pl.ds(chunk_base, _NCHUNK)], idx_u)
    pltpu.sync_copy(item_i_h.at[pl.ds(chunk_base, _NCHUNK)], idx_i)
    pltpu.sync_copy(item_j_h.at[pl.ds(chunk_base, _NCHUNK)], idx_j)

    copies = []
    for ch in range(_NCHUNK):
        dst = pl.ds(ch * _CHUNK, _CHUNK)
        copies.append(pltpu.async_copy(uw_h.at[idx_u.at[ch]], rows_u.at[dst], sem))
        copies.append(pltpu.async_copy(iw_h.at[idx_i.at[ch]], rows_i.at[dst], sem))
        copies.append(pltpu.async_copy(iw_h.at[idx_j.at[ch]], rows_j.at[dst], sem))
    for cp in copies:
        cp.wait()

    lanes = lax.iota(jnp.int32, _LANES)

    def group_body(g, _):
        acc_i = jnp.zeros((_LANES,), jnp.float32)
        acc_j = jnp.zeros((_LANES,), jnp.float32)
        for k in range(_LANES):
            r = g * _LANES + k
            u0 = rows_u[r, pl.ds(0, _LANES)]
            pi = u0 * rows_i[r, pl.ds(0, _LANES)]
            pj = u0 * rows_j[r, pl.ds(0, _LANES)]
            for q in range(1, _D // _LANES):
                sl = pl.ds(q * _LANES, _LANES)
                uq = rows_u[r, sl]
                pi = pi + uq * rows_i[r, sl]
                pj = pj + uq * rows_j[r, sl]
            acc_i = jnp.where(lanes == k, jnp.sum(pi), acc_i)
            acc_j = jnp.where(lanes == k, jnp.sum(pj), acc_j)
        out_i_v[pl.ds(g * _LANES, _LANES)] = acc_i
        out_j_v[pl.ds(g * _LANES, _LANES)] = acc_j
        return 0

    lax.fori_loop(0, _BPW // _LANES, group_body, 0)

    pltpu.sync_copy(out_i_v, out_i_h.at[pl.ds(base, _BPW)])
    pltpu.sync_copy(out_j_v, out_j_h.at[pl.ds(base, _BPW)])


_bpr_call = pl.kernel(
    _bpr_body,
    out_type=(
        jax.ShapeDtypeStruct((_B,), jnp.float32),
        jax.ShapeDtypeStruct((_B,), jnp.float32),
    ),
    mesh=plsc.VectorSubcoreMesh(
        core_axis_name="c", subcore_axis_name="s",
        num_cores=_NC, num_subcores=_NS,
    ),
    compiler_params=pltpu.CompilerParams(
        needs_layout_passes=False, use_tc_tiling_on_sc=False),
    scratch_types=[
        pltpu.VMEM((_NCHUNK, _CHUNK), jnp.int32),
        pltpu.VMEM((_NCHUNK, _CHUNK), jnp.int32),
        pltpu.VMEM((_NCHUNK, _CHUNK), jnp.int32),
        pltpu.VMEM((_BPW, _D), jnp.float32),
        pltpu.VMEM((_BPW, _D), jnp.float32),
        pltpu.VMEM((_BPW, _D), jnp.float32),
        pltpu.VMEM((_BPW,), jnp.float32),
        pltpu.VMEM((_BPW,), jnp.float32),
        pltpu.SemaphoreType.DMA,
    ],
)


def kernel(user, item_i, item_j, embed_user_weight, embed_item_weight):
    shape2d = (_B // _CHUNK, _CHUNK)
    u2 = user.astype(jnp.int32).reshape(shape2d)
    i2 = item_i.astype(jnp.int32).reshape(shape2d)
    j2 = item_j.astype(jnp.int32).reshape(shape2d)
    return _bpr_call(u2, i2, j2, embed_user_weight, embed_item_weight)
